# Initial kernel scaffold; baseline (speedup 1.0000x reference)
import jax
import jax.numpy as jnp
from jax.experimental import pallas as pl


def kernel(input_features, query_weight):
    raise NotImplementedError("write your pallas kernel here")


# speedup vs baseline: 4.6695x; 4.6695x over previous
import functools

import jax
import jax.numpy as jnp
from jax.experimental import pallas as pl

_B, _C, _Q, _F = 8, 192, 256, 1024
_K = 192
_M = _Q - _K + 1


def _attn_kernel(qw_ref, x_ref, out_ref):
    x = x_ref[0]
    xn = x / jnp.maximum(jnp.sqrt(jnp.sum(x * x, axis=0, keepdims=True)), 1e-12)

    qw = qw_ref[...]
    qn = qw / jnp.maximum(jnp.sqrt(jnp.sum(qw * qw, axis=1, keepdims=True)), 1e-12)

    om = jnp.dot(qn, xn, preferred_element_type=jnp.float32)

    i = jax.lax.bitcast_convert_type(om, jnp.int32)
    key = i ^ (jax.lax.shift_right_arithmetic(i, 31) & jnp.int32(0x7FFFFFFF))

    a = jnp.full((1, _F), jnp.iinfo(jnp.int32).min, dtype=jnp.int32)
    cnt = jnp.sum((key >= 0).astype(jnp.int32), axis=0, keepdims=True)
    a = jnp.where(cnt >= _M, jnp.int32(0), a)
    for bit in range(30, -1, -1):
        c = a + jnp.int32(1 << bit)
        cnt = jnp.sum((key >= c).astype(jnp.int32), axis=0, keepdims=True)
        a = jnp.where(cnt >= _M, c, a)

    keep = key >= a
    logits = jnp.where(keep, om, jnp.float32(-1e30))
    mx = jnp.max(logits, axis=1, keepdims=True)
    e = jnp.exp(logits - mx)
    sm = e / jnp.sum(e, axis=1, keepdims=True)

    out_ref[0] = jax.lax.dot_general(
        sm, xn, (((1,), (1,)), ((), ())), preferred_element_type=jnp.float32)


@jax.jit
def kernel(input_features, query_weight):
    x = input_features.reshape(_B, _C, _F)
    fn = pl.pallas_call(
        _attn_kernel,
        grid=(_B,),
        in_specs=[
            pl.BlockSpec((_Q, _C), lambda b: (0, 0)),
            pl.BlockSpec((1, _C, _F), lambda b: (b, 0, 0)),
        ],
        out_specs=pl.BlockSpec((1, _Q, _C), lambda b: (b, 0, 0)),
        out_shape=jax.ShapeDtypeStruct((_B, _Q, _C), jnp.float32),
    )
    return fn(query_weight, x)


# speedup vs baseline: 4.7789x; 1.0234x over previous
import functools

import jax
import jax.numpy as jnp
from jax.experimental import pallas as pl

_B, _C, _Q, _F = 8, 192, 256, 1024
_K = 192
_M = _Q - _K + 1


def _attn_kernel(qw_ref, x_ref, out_ref):
    x = x_ref[0]
    xn = x / jnp.maximum(jnp.sqrt(jnp.sum(x * x, axis=0, keepdims=True)), 1e-12)

    qw = qw_ref[...]
    qn = qw / jnp.maximum(jnp.sqrt(jnp.sum(qw * qw, axis=1, keepdims=True)), 1e-12)

    om = jnp.dot(qn, xn, preferred_element_type=jnp.float32)

    i = jax.lax.bitcast_convert_type(om, jnp.int32)
    key = i ^ (jax.lax.shift_right_arithmetic(i, 31) & jnp.int32(0x7FFFFFFF))

    ones = jnp.ones((1, _Q), dtype=jnp.float32)

    def _count_ge(c):
        ind = jnp.where(key >= c, 1.0, 0.0).astype(jnp.float32)
        return jnp.dot(ones, ind, preferred_element_type=jnp.float32)

    m = jnp.float32(_M)
    cnt = _count_ge(jnp.zeros((1, _F), jnp.int32))
    a = jnp.where(cnt >= m, jnp.int32(0),
                  jnp.int32(jnp.iinfo(jnp.int32).min) + jnp.int32(1 << 30))
    a = jnp.broadcast_to(a, (1, _F))
    for bit in range(29, -1, -1):
        c = a + jnp.int32(1 << bit)
        a = jnp.where(_count_ge(c) >= m, c, a)

    keep = key >= a
    logits = jnp.where(keep, om, jnp.float32(-1e30))
    mx = jnp.max(logits, axis=1, keepdims=True)
    e = jnp.exp(logits - mx)
    sm = e / jnp.sum(e, axis=1, keepdims=True)

    out_ref[0] = jax.lax.dot_general(
        sm, xn, (((1,), (1,)), ((), ())), preferred_element_type=jnp.float32)


@jax.jit
def kernel(input_features, query_weight):
    x = input_features.reshape(_B, _C, _F)
    fn = pl.pallas_call(
        _attn_kernel,
        grid=(_B,),
        in_specs=[
            pl.BlockSpec((_Q, _C), lambda b: (0, 0)),
            pl.BlockSpec((1, _C, _F), lambda b: (b, 0, 0)),
        ],
        out_specs=pl.BlockSpec((1, _Q, _C), lambda b: (b, 0, 0)),
        out_shape=jax.ShapeDtypeStruct((_B, _Q, _C), jnp.float32),
    )
    return fn(query_weight, x)
